# baseline (speedup 1.0000x reference)
import jax
import jax.numpy as jnp
from jax import lax
from jax.experimental import pallas as pl
from jax.experimental.pallas import tpu as pltpu
from jax.experimental.pallas import tpu_sc as plsc
import functools

NC = 2
NS = 16
NW = NC * NS

IDX_ROWS = 25600
ROWS_PER_W = IDX_ROWS // NW
KCH = 4
N_ITER = ROWS_PER_W // KCH


def _sc_body(w_hbm, idx_hbm, out_hbm, idx_v, rows_v, sem):
    wid = lax.axis_index("s") * NC + lax.axis_index("c")
    base_row = wid * ROWS_PER_W

    def step(i, _):
        r0 = base_row + i * KCH
        pltpu.sync_copy(idx_hbm.at[pl.ds(r0, KCH)], idx_v)
        copies = [
            pltpu.async_copy(
                w_hbm.at[idx_v.at[k]],
                rows_v.at[pl.ds(k * 128, 128)],
                sem,
            )
            for k in range(KCH)
        ]
        for c in copies:
            c.wait()
        pltpu.sync_copy(rows_v, out_hbm.at[pl.ds(r0 * 128, KCH * 128)])
        return _

    lax.fori_loop(0, N_ITER, step, 0)


@functools.partial(jax.jit, static_argnames=())
def kernel(x, W):
    x_flat = x.astype(jnp.int32).reshape(IDX_ROWS, 128)
    mesh = plsc.VectorSubcoreMesh(
        core_axis_name="c", subcore_axis_name="s",
        num_cores=NC, num_subcores=NS,
    )
    out_flat = pl.kernel(
        _sc_body,
        out_type=jax.ShapeDtypeStruct((IDX_ROWS * 128, 128), jnp.float32),
        mesh=mesh,
        scratch_types=[
            pltpu.VMEM((KCH, 128), jnp.int32),
            pltpu.VMEM((KCH * 128, 128), jnp.float32),
            pltpu.SemaphoreType.DMA,
        ],
    )(W, x_flat)
    return out_flat.reshape(x.shape[0], x.shape[1], 128)


# speedup vs baseline: 54.5564x; 54.5564x over previous
import jax
import jax.numpy as jnp
from jax import lax
from jax.experimental import pallas as pl
from jax.experimental.pallas import tpu as pltpu
from jax.experimental.pallas import tpu_sc as plsc
import functools

NC = 2
NS = 16
NW = NC * NS
L = 16
D = 128

N_IDX = 16384 * 200
ROWS_PER_W = N_IDX // NW
CB = 512
N_CHUNK = ROWS_PER_W // CB


_SPLAT_DN = lax.GatherDimensionNumbers(
    offset_dims=(), collapsed_slice_dims=(0,), start_index_map=(0,))


def _splat(vec, j):
    idx = jnp.full((L, 1), j, jnp.int32)
    return lax.gather(vec, idx, _SPLAT_DN, slice_sizes=(1,),
                      mode=lax.GatherScatterMode.PROMISE_IN_BOUNDS)


def _sc_body(w_hbm, idx_hbm, out_hbm, w_v, idx_v, rows_v, sem):
    wid = lax.axis_index("s") * NC + lax.axis_index("c")
    base = wid * ROWS_PER_W

    pltpu.sync_copy(w_hbm, w_v)
    w0 = [w_v[pl.ds(k * L, L)] for k in range(8)]
    dw = [w_v[pl.ds(D + k * L, L)] - w0[k] for k in range(8)]

    def chunk(i, _):
        r0 = base + i * CB
        pltpu.sync_copy(idx_hbm.at[pl.ds(r0, CB)], idx_v)

        @plsc.parallel_loop(0, CB // L, 1, unroll=2)
        def group(g):
            xvf = idx_v[pl.ds(g * L, L)].astype(jnp.float32)
            for j in range(L):
                xs = _splat(xvf, j)
                r = g * L + j
                for k in range(8):
                    rows_v[pl.ds(r * D + k * L, L)] = w0[k] + xs * dw[k]

        pltpu.sync_copy(rows_v, out_hbm.at[pl.ds(r0 * D, CB * D)])
        return _

    lax.fori_loop(0, N_CHUNK, chunk, 0)


@functools.partial(jax.jit, static_argnames=())
def kernel(x, W):
    x_flat = x.astype(jnp.int32).reshape(N_IDX)
    w_flat = W.reshape(2 * D)
    mesh = plsc.VectorSubcoreMesh(
        core_axis_name="c", subcore_axis_name="s",
        num_cores=NC, num_subcores=NS,
    )
    out_flat = pl.kernel(
        _sc_body,
        out_type=jax.ShapeDtypeStruct((N_IDX * D,), jnp.float32),
        mesh=mesh,
        scratch_types=[
            pltpu.VMEM((2 * D,), jnp.float32),
            pltpu.VMEM((CB,), jnp.int32),
            pltpu.VMEM((CB * D,), jnp.float32),
            pltpu.SemaphoreType.DMA,
        ],
    )(w_flat, x_flat)
    return out_flat.reshape(x.shape[0], x.shape[1], D)


# speedup vs baseline: 108.2309x; 1.9838x over previous
import jax
import jax.numpy as jnp
from jax import lax
from jax.experimental import pallas as pl
from jax.experimental.pallas import tpu as pltpu
from jax.experimental.pallas import tpu_sc as plsc
import functools

NC = 2
NS = 16
NW = NC * NS
L = 16
D = 128

N_IDX = 16384 * 200
ROWS_PER_W = N_IDX // NW
CB = 256
N_CHUNK = ROWS_PER_W // CB
IB = 4096
CH_PER_IB = IB // CB

_SPLAT_DN = lax.GatherDimensionNumbers(
    offset_dims=(), collapsed_slice_dims=(0,), start_index_map=(0,))


def _splat(vec, j):
    idx = jnp.full((L, 1), j, jnp.int32)
    return lax.gather(vec, idx, _SPLAT_DN, slice_sizes=(1,),
                      mode=lax.GatherScatterMode.PROMISE_IN_BOUNDS)


def _sc_body(w_hbm, idx_hbm, out_hbm, w_v, idx_v, rows_a, rows_b, sems):
    wid = lax.axis_index("s") * NC + lax.axis_index("c")
    base = wid * ROWS_PER_W

    pltpu.sync_copy(w_hbm, w_v)
    w0 = [w_v[pl.ds(k * L, L)] for k in range(8)]
    dw = [w_v[pl.ds(D + k * L, L)] - w0[k] for k in range(8)]

    def compute_chunk(rows_v, idx_off):
        @plsc.parallel_loop(0, CB // L, 1, unroll=2)
        def group(g):
            xvf = idx_v[pl.ds(idx_off + g * L, L)].astype(jnp.float32)
            for j in range(L):
                xs = _splat(xvf, j)
                r = g * L + j
                for k in range(8):
                    rows_v[pl.ds(r * D + k * L, L)] = w0[k] + xs * dw[k]

    def pair(i, _):
        for b, rows_v in ((0, rows_a), (1, rows_b)):
            c = 2 * i + b
            r0 = base + c * CB
            if b == 0:
                @pl.when(lax.rem(i, CH_PER_IB // 2) == 0)
                def _load_idx():
                    slab = i // (CH_PER_IB // 2)
                    pltpu.sync_copy(
                        idx_hbm.at[pl.ds(base + slab * IB, IB)], idx_v)

            @pl.when(i > 0)
            def _drain():
                pltpu.make_async_copy(
                    rows_v,
                    out_hbm.at[pl.ds((r0 - 2 * CB) * D, CB * D)],
                    sems.at[b],
                ).wait()

            compute_chunk(rows_v, lax.rem(c, CH_PER_IB) * CB)
            pltpu.async_copy(
                rows_v, out_hbm.at[pl.ds(r0 * D, CB * D)], sems.at[b])
        return _

    lax.fori_loop(0, N_CHUNK // 2, pair, 0)

    last = base + N_CHUNK * CB
    for b, rows_v in ((0, rows_a), (1, rows_b)):
        pltpu.make_async_copy(
            rows_v,
            out_hbm.at[pl.ds((last - (2 - b) * CB) * D, CB * D)],
            sems.at[b],
        ).wait()


@functools.partial(jax.jit, static_argnames=())
def kernel(x, W):
    x_flat = x.astype(jnp.int32).reshape(N_IDX)
    w_flat = W.reshape(2 * D)
    mesh = plsc.VectorSubcoreMesh(
        core_axis_name="c", subcore_axis_name="s",
        num_cores=NC, num_subcores=NS,
    )
    out_flat = pl.kernel(
        _sc_body,
        out_type=jax.ShapeDtypeStruct((N_IDX * D,), jnp.float32),
        mesh=mesh,
        scratch_types=[
            pltpu.VMEM((2 * D,), jnp.float32),
            pltpu.VMEM((IB,), jnp.int32),
            pltpu.VMEM((CB * D,), jnp.float32),
            pltpu.VMEM((CB * D,), jnp.float32),
            pltpu.SemaphoreType.DMA((2,)),
        ],
    )(w_flat, x_flat)
    return out_flat.reshape(x.shape[0], x.shape[1], D)
